# baseline (speedup 1.0000x reference)
import functools

import jax
import jax.numpy as jnp
from jax import lax
from jax.experimental import pallas as pl
from jax.experimental.pallas import tpu as pltpu
from jax.experimental.pallas import tpu_sc as plsc

N = 10000
E = 160000
D = 256
H = 256
OUT = 128
NC = 3
NG = 64

HALF = 128
NSC = 2
NTILES = 16
EPT = E // NTILES
K = 40
NCH = EPT // K
ACC_ROWS = N
NBUF = 7
STRIPE = 624
TAIL = N - STRIPE * NTILES

BR = 2000
NBLK = N // BR


def _sc_agg_body(h2, src2, dst2, u2, sidx, didx, bufs, acc, *sems):
    c = lax.axis_index("c")
    s = lax.axis_index("s")
    row0 = s * STRIPE
    hrow0 = c * N + row0
    pltpu.sync_copy(h2.at[pl.ds(hrow0, STRIPE)], acc.at[pl.ds(row0, STRIPE)])

    @pl.when(s == 0)
    def _():
        pltpu.sync_copy(h2.at[pl.ds(c * N + STRIPE * NTILES, TAIL)],
                        acc.at[pl.ds(STRIPE * NTILES, TAIL)])
    pltpu.sync_copy(src2.at[c, s], sidx)
    plsc.subcore_barrier()
    gsems = sems[:NBUF]
    dsems = sems[NBUF:2 * NBUF]
    for b in range(NBUF):
        pltpu.async_copy(h2.at[sidx.at[pl.ds(b * K, K)]], bufs.at[b],
                         gsems[b])
        pltpu.async_copy(dst2.at[s, b], didx.at[b], dsems[b])

    def body(g, carry):
        for b in range(NBUF):
            j = g * NBUF + b

            @pl.when(j < NCH)
            def _():
                pltpu.make_async_copy(h2.at[sidx.at[pl.ds(j * K, K)]],
                                      bufs.at[b], gsems[b]).wait()
                pltpu.make_async_copy(dst2.at[s, j], didx.at[b],
                                      dsems[b]).wait()
                pltpu.sync_copy(bufs.at[b], acc.at[didx.at[b]], add=True)
                nj = j + NBUF

                @pl.when(nj < NCH)
                def _():
                    pltpu.async_copy(h2.at[sidx.at[pl.ds(nj * K, K)]],
                                     bufs.at[b], gsems[b])
                    pltpu.async_copy(dst2.at[s, nj], didx.at[b], dsems[b])
        return carry

    lax.fori_loop(0, NCH // NBUF + 1, body, 0)
    plsc.subcore_barrier()
    pltpu.sync_copy(acc.at[pl.ds(row0, STRIPE)], u2.at[pl.ds(hrow0, STRIPE)])

    @pl.when(s == 0)
    def _():
        pltpu.sync_copy(acc.at[pl.ds(STRIPE * NTILES, TAIL)],
                        u2.at[pl.ds(c * N + STRIPE * NTILES, TAIL)])


@functools.cache
def _sc_agg():
    return pl.kernel(
        _sc_agg_body,
        out_type=jax.ShapeDtypeStruct((NSC * N, HALF), jnp.float32),
        mesh=plsc.VectorSubcoreMesh(core_axis_name="c", subcore_axis_name="s"),
        scratch_types=[
            pltpu.VMEM((EPT,), jnp.int32),
            pltpu.VMEM((NBUF, K), jnp.int32),
            pltpu.VMEM((NBUF, K, HALF), jnp.float32),
            pltpu.VMEM_SHARED((ACC_ROWS, HALF), jnp.float32),
        ] + [pltpu.SemaphoreType.DMA] * (2 * NBUF),
    )


def _mlp_body(ul_ref, ur_ref, w1_ref, b1_ref, g_ref, be_ref, w2_ref, b2_ref,
              batch_ref, z_ref, pooled_ref, t_vmem, stats_vmem):
    _mlp_common(ul_ref, ur_ref, w1_ref, b1_ref, g_ref, be_ref, w2_ref, b2_ref,
                batch_ref, z_ref, pooled_ref, t_vmem, stats_vmem)


def _mlp_head_body(ul_ref, ur_ref, w1_ref, b1_ref, g_ref, be_ref, w2_ref,
                   b2_ref, batch_ref, p0_ref, p1_ref, wl_ref, bl_ref,
                   z_ref, pooled_ref, o_ref, t_vmem, stats_vmem):
    _mlp_common(ul_ref, ur_ref, w1_ref, b1_ref, g_ref, be_ref, w2_ref, b2_ref,
                batch_ref, z_ref, pooled_ref, t_vmem, stats_vmem)

    @pl.when((pl.program_id(0) == 1) & (pl.program_id(1) == NBLK - 1))
    def _():
        acc = jnp.dot(p0_ref[...], wl_ref[0:H, :],
                      preferred_element_type=jnp.float32)
        acc = acc + jnp.dot(p1_ref[...], wl_ref[H:2 * H, :],
                            preferred_element_type=jnp.float32)
        acc = acc + jnp.dot(pooled_ref[...], wl_ref[2 * H:, :],
                            preferred_element_type=jnp.float32)
        acc = acc + bl_ref[...]
        o_ref[...] = 1.0 / (1.0 + jnp.exp(-acc))


def _mlp_common(ul_ref, ur_ref, w1_ref, b1_ref, g_ref, be_ref, w2_ref, b2_ref,
                batch_ref, z_ref, pooled_ref, t_vmem, stats_vmem):
    ph = pl.program_id(0)
    i = pl.program_id(1)

    @pl.when(ph == 0)
    def _():
        t = jnp.dot(ul_ref[...], w1_ref[:HALF, :],
                    preferred_element_type=jnp.float32)
        t = t + jnp.dot(ur_ref[...], w1_ref[HALF:, :],
                        preferred_element_type=jnp.float32)
        t = t + b1_ref[...]
        t_vmem[pl.ds(i * BR, BR), :] = t.astype(jnp.bfloat16)

        @pl.when(i == 0)
        def _():
            stats_vmem[...] = jnp.zeros_like(stats_vmem)

        s1 = jnp.sum(t, axis=0, keepdims=True)
        s2 = jnp.sum(t * t, axis=0, keepdims=True)
        stats_vmem[0:1, :] = stats_vmem[0:1, :] + s1
        stats_vmem[1:2, :] = stats_vmem[1:2, :] + s2

    @pl.when(ph == 1)
    def _():
        t = t_vmem[pl.ds(i * BR, BR), :].astype(jnp.float32)
        mu = stats_vmem[0:1, :] * (1.0 / N)
        msq = stats_vmem[1:2, :] * (1.0 / N)
        var = msq - mu * mu
        scale = g_ref[...] * lax.rsqrt(var + 1e-5)
        shift = be_ref[...] - mu * scale
        hbn = jnp.maximum(t * scale + shift, 0.0)
        z = jnp.dot(hbn, w2_ref[...], preferred_element_type=jnp.float32)
        z = jnp.maximum(z + b2_ref[...], 0.0)
        z_ref[0] = z[:, :HALF]
        z_ref[1] = z[:, HALF:]
        onehot = (lax.broadcasted_iota(jnp.int32, (NG, BR), 0)
                  .astype(jnp.float32) == batch_ref[0]).astype(jnp.float32)
        p = jnp.dot(onehot, z, preferred_element_type=jnp.float32)

        @pl.when(i == 0)
        def _():
            pooled_ref[...] = jnp.zeros_like(pooled_ref)

        pooled_ref[...] = pooled_ref[...] + p


_MLP_IN_SPECS = [
    pl.BlockSpec((BR, HALF), lambda p, i: (jnp.where(p == 0, i, 0), 0)),
    pl.BlockSpec((BR, HALF),
                 lambda p, i: (NBLK + jnp.where(p == 0, i, 0), 0)),
    pl.BlockSpec((H, H), lambda p, i: (0, 0)),
    pl.BlockSpec((1, H), lambda p, i: (0, 0)),
    pl.BlockSpec((1, H), lambda p, i: (0, 0)),
    pl.BlockSpec((1, H), lambda p, i: (0, 0)),
    pl.BlockSpec((H, H), lambda p, i: (0, 0)),
    pl.BlockSpec((1, H), lambda p, i: (0, 0)),
    pl.BlockSpec((1, 1, BR), lambda p, i: (i, 0, 0)),
]
_MLP_SCRATCH = [
    pltpu.VMEM((N, H), jnp.bfloat16),
    pltpu.VMEM((8, H), jnp.float32),
]

_mlp = pl.pallas_call(
    _mlp_body,
    grid=(2, NBLK),
    in_specs=list(_MLP_IN_SPECS),
    out_specs=[
        pl.BlockSpec((2, BR, HALF), lambda p, i: (0, jnp.where(p == 1, i, 0), 0)),
        pl.BlockSpec((NG, H), lambda p, i: (0, 0)),
    ],
    out_shape=[
        jax.ShapeDtypeStruct((2, N, HALF), jnp.float32),
        jax.ShapeDtypeStruct((NG, H), jnp.float32),
    ],
    scratch_shapes=list(_MLP_SCRATCH),
)

_mlp_head = pl.pallas_call(
    _mlp_head_body,
    grid=(2, NBLK),
    in_specs=list(_MLP_IN_SPECS) + [
        pl.BlockSpec((NG, H), lambda p, i: (0, 0)),
        pl.BlockSpec((NG, H), lambda p, i: (0, 0)),
        pl.BlockSpec((NC * H, OUT), lambda p, i: (0, 0)),
        pl.BlockSpec((1, OUT), lambda p, i: (0, 0)),
    ],
    out_specs=[
        pl.BlockSpec((2, BR, HALF), lambda p, i: (0, jnp.where(p == 1, i, 0), 0)),
        pl.BlockSpec((NG, H), lambda p, i: (0, 0)),
        pl.BlockSpec((NG, OUT), lambda p, i: (0, 0)),
    ],
    out_shape=[
        jax.ShapeDtypeStruct((2, N, HALF), jnp.float32),
        jax.ShapeDtypeStruct((NG, H), jnp.float32),
        jax.ShapeDtypeStruct((NG, OUT), jnp.float32),
    ],
    scratch_shapes=list(_MLP_SCRATCH),
)


def kernel(x, edge_index, batch,
           W1_0, b1_0, g_0, be_0, W2_0, b2_0,
           W1_1, b1_1, g_1, be_1, W2_1, b2_1,
           W1_2, b1_2, g_2, be_2, W2_2, b2_2,
           Wl, bl):
    src = edge_index[0].astype(jnp.int32)
    dst = edge_index[1].astype(jnp.int32)
    src2 = jnp.stack([src, src + N]).reshape(NSC, NTILES, EPT)
    dst2 = dst.reshape(NTILES, NCH, K)
    batchf = batch.astype(jnp.float32).reshape(NBLK, 1, BR)

    h2 = jnp.concatenate([x[:, :HALF], x[:, HALF:]], axis=0)
    params = [
        (W1_0, b1_0, g_0, be_0, W2_0, b2_0),
        (W1_1, b1_1, g_1, be_1, W2_1, b2_1),
        (W1_2, b1_2, g_2, be_2, W2_2, b2_2),
    ]
    pooled = []
    for (W1, b1, g, be, W2, b2) in params[:2]:
        u2 = _sc_agg()(h2, src2, dst2)
        z2, p = _mlp(u2, u2, W1, b1.reshape(1, H),
                     g.reshape(1, H), be.reshape(1, H), W2,
                     b2.reshape(1, H), batchf)
        h2 = z2.reshape(NSC * N, HALF)
        pooled.append(p)
    (W1, b1, g, be, W2, b2) = params[2]
    u2 = _sc_agg()(h2, src2, dst2)
    _, _, out = _mlp_head(u2, u2, W1, b1.reshape(1, H),
                          g.reshape(1, H), be.reshape(1, H), W2,
                          b2.reshape(1, H), batchf,
                          pooled[0], pooled[1], Wl, bl.reshape(1, OUT))
    return out


# speedup vs baseline: 1.0115x; 1.0115x over previous
import functools

import jax
import jax.numpy as jnp
from jax import lax
from jax.experimental import pallas as pl
from jax.experimental.pallas import tpu as pltpu
from jax.experimental.pallas import tpu_sc as plsc

N = 10000
E = 160000
D = 256
H = 256
OUT = 128
NC = 3
NG = 64

HALF = 128
NSC = 2
NTILES = 16
EPT = E // NTILES
K = 40
NCH = EPT // K
ACC_ROWS = N
NBUF = 6
STRIPE = 624
TAIL = N - STRIPE * NTILES

BR = 2000
NBLK = N // BR


def _sc_agg_body(h2, src2, dst2, u2, sidx, didx, bufs, acc, *sems):
    c = lax.axis_index("c")
    s = lax.axis_index("s")
    row0 = s * STRIPE
    hrow0 = c * N + row0
    pltpu.sync_copy(h2.at[pl.ds(hrow0, STRIPE)], acc.at[pl.ds(row0, STRIPE)])

    @pl.when(s == 0)
    def _():
        pltpu.sync_copy(h2.at[pl.ds(c * N + STRIPE * NTILES, TAIL)],
                        acc.at[pl.ds(STRIPE * NTILES, TAIL)])
    pltpu.sync_copy(src2.at[c, s], sidx)
    plsc.subcore_barrier()
    gsems = sems[:NBUF]
    dsems = sems[NBUF:2 * NBUF]
    for b in range(NBUF):
        pltpu.async_copy(h2.at[sidx.at[pl.ds(b * K, K)]], bufs.at[b],
                         gsems[b])
        pltpu.async_copy(dst2.at[s, b], didx.at[b], dsems[b])

    def body(g, carry):
        for b in range(NBUF):
            j = g * NBUF + b

            @pl.when(j < NCH)
            def _():
                pltpu.make_async_copy(h2.at[sidx.at[pl.ds(j * K, K)]],
                                      bufs.at[b], gsems[b]).wait()
                pltpu.make_async_copy(dst2.at[s, j], didx.at[b],
                                      dsems[b]).wait()
                pltpu.sync_copy(bufs.at[b], acc.at[didx.at[b]], add=True)
                nj = j + NBUF

                @pl.when(nj < NCH)
                def _():
                    pltpu.async_copy(h2.at[sidx.at[pl.ds(nj * K, K)]],
                                     bufs.at[b], gsems[b])
                    pltpu.async_copy(dst2.at[s, nj], didx.at[b], dsems[b])
        return carry

    lax.fori_loop(0, NCH // NBUF + 1, body, 0)
    plsc.subcore_barrier()
    pltpu.sync_copy(acc.at[pl.ds(row0, STRIPE)], u2.at[pl.ds(hrow0, STRIPE)])

    @pl.when(s == 0)
    def _():
        pltpu.sync_copy(acc.at[pl.ds(STRIPE * NTILES, TAIL)],
                        u2.at[pl.ds(c * N + STRIPE * NTILES, TAIL)])


@functools.cache
def _sc_agg():
    return pl.kernel(
        _sc_agg_body,
        out_type=jax.ShapeDtypeStruct((NSC * N, HALF), jnp.float32),
        mesh=plsc.VectorSubcoreMesh(core_axis_name="c", subcore_axis_name="s"),
        scratch_types=[
            pltpu.VMEM((EPT,), jnp.int32),
            pltpu.VMEM((NBUF, K), jnp.int32),
            pltpu.VMEM((NBUF, K, HALF), jnp.float32),
            pltpu.VMEM_SHARED((ACC_ROWS, HALF), jnp.float32),
        ] + [pltpu.SemaphoreType.DMA] * (2 * NBUF),
    )


def _mlp_body(ul_ref, ur_ref, w1_ref, b1_ref, g_ref, be_ref, w2_ref, b2_ref,
              batch_ref, z_ref, pooled_ref, t_vmem, stats_vmem):
    _mlp_common(ul_ref, ur_ref, w1_ref, b1_ref, g_ref, be_ref, w2_ref, b2_ref,
                batch_ref, z_ref, pooled_ref, t_vmem, stats_vmem)


def _mlp_head_body(ul_ref, ur_ref, w1_ref, b1_ref, g_ref, be_ref, w2_ref,
                   b2_ref, batch_ref, p0_ref, p1_ref, wl_ref, bl_ref,
                   z_ref, pooled_ref, o_ref, t_vmem, stats_vmem):
    _mlp_common(ul_ref, ur_ref, w1_ref, b1_ref, g_ref, be_ref, w2_ref, b2_ref,
                batch_ref, z_ref, pooled_ref, t_vmem, stats_vmem)

    @pl.when((pl.program_id(0) == 1) & (pl.program_id(1) == NBLK - 1))
    def _():
        acc = jnp.dot(p0_ref[...], wl_ref[0:H, :],
                      preferred_element_type=jnp.float32)
        acc = acc + jnp.dot(p1_ref[...], wl_ref[H:2 * H, :],
                            preferred_element_type=jnp.float32)
        acc = acc + jnp.dot(pooled_ref[...], wl_ref[2 * H:, :],
                            preferred_element_type=jnp.float32)
        acc = acc + bl_ref[...]
        o_ref[...] = 1.0 / (1.0 + jnp.exp(-acc))


def _mlp_common(ul_ref, ur_ref, w1_ref, b1_ref, g_ref, be_ref, w2_ref, b2_ref,
                batch_ref, z_ref, pooled_ref, t_vmem, stats_vmem):
    ph = pl.program_id(0)
    i = pl.program_id(1)

    @pl.when(ph == 0)
    def _():
        t = jnp.dot(ul_ref[...], w1_ref[:HALF, :],
                    preferred_element_type=jnp.float32)
        t = t + jnp.dot(ur_ref[...], w1_ref[HALF:, :],
                        preferred_element_type=jnp.float32)
        t = t + b1_ref[...]
        t_vmem[pl.ds(i * BR, BR), :] = t.astype(jnp.bfloat16)

        @pl.when(i == 0)
        def _():
            stats_vmem[...] = jnp.zeros_like(stats_vmem)

        s1 = jnp.sum(t, axis=0, keepdims=True)
        s2 = jnp.sum(t * t, axis=0, keepdims=True)
        stats_vmem[0:1, :] = stats_vmem[0:1, :] + s1
        stats_vmem[1:2, :] = stats_vmem[1:2, :] + s2

    @pl.when(ph == 1)
    def _():
        t = t_vmem[pl.ds(i * BR, BR), :].astype(jnp.float32)
        mu = stats_vmem[0:1, :] * (1.0 / N)
        msq = stats_vmem[1:2, :] * (1.0 / N)
        var = msq - mu * mu
        scale = g_ref[...] * lax.rsqrt(var + 1e-5)
        shift = be_ref[...] - mu * scale
        hbn = jnp.maximum(t * scale + shift, 0.0)
        z = jnp.dot(hbn, w2_ref[...], preferred_element_type=jnp.float32)
        z = jnp.maximum(z + b2_ref[...], 0.0)
        z_ref[0] = z[:, :HALF]
        z_ref[1] = z[:, HALF:]
        onehot = (lax.broadcasted_iota(jnp.int32, (NG, BR), 0)
                  .astype(jnp.float32) == batch_ref[0]).astype(jnp.float32)
        p = jnp.dot(onehot, z, preferred_element_type=jnp.float32)

        @pl.when(i == 0)
        def _():
            pooled_ref[...] = jnp.zeros_like(pooled_ref)

        pooled_ref[...] = pooled_ref[...] + p


_MLP_IN_SPECS = [
    pl.BlockSpec((BR, HALF), lambda p, i: (jnp.where(p == 0, i, 0), 0)),
    pl.BlockSpec((BR, HALF),
                 lambda p, i: (NBLK + jnp.where(p == 0, i, 0), 0)),
    pl.BlockSpec((H, H), lambda p, i: (0, 0)),
    pl.BlockSpec((1, H), lambda p, i: (0, 0)),
    pl.BlockSpec((1, H), lambda p, i: (0, 0)),
    pl.BlockSpec((1, H), lambda p, i: (0, 0)),
    pl.BlockSpec((H, H), lambda p, i: (0, 0)),
    pl.BlockSpec((1, H), lambda p, i: (0, 0)),
    pl.BlockSpec((1, 1, BR), lambda p, i: (i, 0, 0)),
]
_MLP_SCRATCH = [
    pltpu.VMEM((N, H), jnp.bfloat16),
    pltpu.VMEM((8, H), jnp.float32),
]

_mlp = pl.pallas_call(
    _mlp_body,
    grid=(2, NBLK),
    in_specs=list(_MLP_IN_SPECS),
    out_specs=[
        pl.BlockSpec((2, BR, HALF), lambda p, i: (0, jnp.where(p == 1, i, 0), 0)),
        pl.BlockSpec((NG, H), lambda p, i: (0, 0)),
    ],
    out_shape=[
        jax.ShapeDtypeStruct((2, N, HALF), jnp.float32),
        jax.ShapeDtypeStruct((NG, H), jnp.float32),
    ],
    scratch_shapes=list(_MLP_SCRATCH),
)

_mlp_head = pl.pallas_call(
    _mlp_head_body,
    grid=(2, NBLK),
    in_specs=list(_MLP_IN_SPECS) + [
        pl.BlockSpec((NG, H), lambda p, i: (0, 0)),
        pl.BlockSpec((NG, H), lambda p, i: (0, 0)),
        pl.BlockSpec((NC * H, OUT), lambda p, i: (0, 0)),
        pl.BlockSpec((1, OUT), lambda p, i: (0, 0)),
    ],
    out_specs=[
        pl.BlockSpec((2, BR, HALF), lambda p, i: (0, jnp.where(p == 1, i, 0), 0)),
        pl.BlockSpec((NG, H), lambda p, i: (0, 0)),
        pl.BlockSpec((NG, OUT), lambda p, i: (0, 0)),
    ],
    out_shape=[
        jax.ShapeDtypeStruct((2, N, HALF), jnp.float32),
        jax.ShapeDtypeStruct((NG, H), jnp.float32),
        jax.ShapeDtypeStruct((NG, OUT), jnp.float32),
    ],
    scratch_shapes=list(_MLP_SCRATCH),
)


def kernel(x, edge_index, batch,
           W1_0, b1_0, g_0, be_0, W2_0, b2_0,
           W1_1, b1_1, g_1, be_1, W2_1, b2_1,
           W1_2, b1_2, g_2, be_2, W2_2, b2_2,
           Wl, bl):
    src = edge_index[0].astype(jnp.int32)
    dst = edge_index[1].astype(jnp.int32)
    src2 = jnp.stack([src, src + N]).reshape(NSC, NTILES, EPT)
    dst2 = dst.reshape(NTILES, NCH, K)
    batchf = batch.astype(jnp.float32).reshape(NBLK, 1, BR)

    h2 = jnp.concatenate([x[:, :HALF], x[:, HALF:]], axis=0)
    params = [
        (W1_0, b1_0, g_0, be_0, W2_0, b2_0),
        (W1_1, b1_1, g_1, be_1, W2_1, b2_1),
        (W1_2, b1_2, g_2, be_2, W2_2, b2_2),
    ]
    pooled = []
    for (W1, b1, g, be, W2, b2) in params[:2]:
        u2 = _sc_agg()(h2, src2, dst2)
        z2, p = _mlp(u2, u2, W1, b1.reshape(1, H),
                     g.reshape(1, H), be.reshape(1, H), W2,
                     b2.reshape(1, H), batchf)
        h2 = z2.reshape(NSC * N, HALF)
        pooled.append(p)
    (W1, b1, g, be, W2, b2) = params[2]
    u2 = _sc_agg()(h2, src2, dst2)
    _, _, out = _mlp_head(u2, u2, W1, b1.reshape(1, H),
                          g.reshape(1, H), be.reshape(1, H), W2,
                          b2.reshape(1, H), batchf,
                          pooled[0], pooled[1], Wl, bl.reshape(1, OUT))
    return out


# speedup vs baseline: 1.0135x; 1.0020x over previous
import functools

import jax
import jax.numpy as jnp
from jax import lax
from jax.experimental import pallas as pl
from jax.experimental.pallas import tpu as pltpu
from jax.experimental.pallas import tpu_sc as plsc

N = 10000
E = 160000
D = 256
H = 256
OUT = 128
NC = 3
NG = 64

HALF = 128
NSC = 2
NTILES = 16
EPT = E // NTILES
K = 40
NCH = EPT // K
ACC_ROWS = N
NBUF = 6
STRIPE = 624
TAIL = N - STRIPE * NTILES

BR = 2000
NBLK = N // BR


def _sc_agg_body(h2, src2, dst2, u2, sidx, didx, bufs, acc, *sems):
    c = lax.axis_index("c")
    s = lax.axis_index("s")
    row0 = s * STRIPE
    hrow0 = c * N + row0
    pltpu.sync_copy(h2.at[pl.ds(hrow0, STRIPE)], acc.at[pl.ds(row0, STRIPE)])

    @pl.when(s == 0)
    def _():
        pltpu.sync_copy(h2.at[pl.ds(c * N + STRIPE * NTILES, TAIL)],
                        acc.at[pl.ds(STRIPE * NTILES, TAIL)])
    pltpu.sync_copy(src2.at[c, s], sidx)
    plsc.subcore_barrier()
    gsems = sems[:NBUF]
    dsems = sems[NBUF:2 * NBUF]
    for b in range(NBUF):
        pltpu.async_copy(h2.at[sidx.at[pl.ds(b * K, K)]], bufs.at[b],
                         gsems[b])
        pltpu.async_copy(dst2.at[s, b], didx.at[b], dsems[b])

    def body(g, carry):
        for b in range(NBUF):
            j = g * NBUF + b

            @pl.when(j < NCH)
            def _():
                pltpu.make_async_copy(h2.at[sidx.at[pl.ds(j * K, K)]],
                                      bufs.at[b], gsems[b]).wait()
                pltpu.make_async_copy(dst2.at[s, j], didx.at[b],
                                      dsems[b]).wait()
                pltpu.sync_copy(bufs.at[b], acc.at[didx.at[b]], add=True)
                nj = j + NBUF

                @pl.when(nj < NCH)
                def _():
                    pltpu.async_copy(h2.at[sidx.at[pl.ds(nj * K, K)]],
                                     bufs.at[b], gsems[b])
                    pltpu.async_copy(dst2.at[s, nj], didx.at[b], dsems[b])
        return carry

    lax.fori_loop(0, NCH // NBUF + 1, body, 0)
    plsc.subcore_barrier()
    pltpu.sync_copy(acc.at[pl.ds(row0, STRIPE)], u2.at[pl.ds(hrow0, STRIPE)])

    @pl.when(s == 0)
    def _():
        pltpu.sync_copy(acc.at[pl.ds(STRIPE * NTILES, TAIL)],
                        u2.at[pl.ds(c * N + STRIPE * NTILES, TAIL)])


@functools.cache
def _sc_agg():
    return pl.kernel(
        _sc_agg_body,
        out_type=jax.ShapeDtypeStruct((NSC * N, HALF), jnp.float32),
        mesh=plsc.VectorSubcoreMesh(core_axis_name="c", subcore_axis_name="s"),
        scratch_types=[
            pltpu.VMEM((EPT,), jnp.int32),
            pltpu.VMEM((NBUF, K), jnp.int32),
            pltpu.VMEM((NBUF, K, HALF), jnp.float32),
            pltpu.VMEM_SHARED((ACC_ROWS, HALF), jnp.float32),
        ] + [pltpu.SemaphoreType.DMA] * (2 * NBUF),
    )


def _mlp_body(ul_ref, ur_ref, w1_ref, b1_ref, g_ref, be_ref, w2_ref, b2_ref,
              batch_ref, z_ref, pooled_ref, t_vmem, stats_vmem):
    _mlp_common(ul_ref, ur_ref, w1_ref, b1_ref, g_ref, be_ref, w2_ref, b2_ref,
                batch_ref, z_ref, pooled_ref, t_vmem, stats_vmem)


def _mlp_head_body(ul_ref, ur_ref, w1_ref, b1_ref, g_ref, be_ref, w2_ref,
                   b2_ref, batch_ref, p0_ref, p1_ref, wl_ref, bl_ref,
                   z_ref, pooled_ref, o_ref, t_vmem, stats_vmem):
    _mlp_common(ul_ref, ur_ref, w1_ref, b1_ref, g_ref, be_ref, w2_ref, b2_ref,
                batch_ref, z_ref, pooled_ref, t_vmem, stats_vmem)

    @pl.when((pl.program_id(0) == 1) & (pl.program_id(1) == NBLK - 1))
    def _():
        acc = jnp.dot(p0_ref[...], wl_ref[0:H, :],
                      preferred_element_type=jnp.float32)
        acc = acc + jnp.dot(p1_ref[...], wl_ref[H:2 * H, :],
                            preferred_element_type=jnp.float32)
        acc = acc + jnp.dot(pooled_ref[...], wl_ref[2 * H:, :],
                            preferred_element_type=jnp.float32)
        acc = acc + bl_ref[...]
        o_ref[...] = 1.0 / (1.0 + jnp.exp(-acc))


def _mlp_common(ul_ref, ur_ref, w1_ref, b1_ref, g_ref, be_ref, w2_ref, b2_ref,
                batch_ref, z_ref, pooled_ref, t_vmem, stats_vmem):
    ph = pl.program_id(0)
    i = pl.program_id(1)

    @pl.when(ph == 0)
    def _():
        t = jnp.dot(ul_ref[...], w1_ref[:HALF, :],
                    preferred_element_type=jnp.float32)
        t = t + jnp.dot(ur_ref[...], w1_ref[HALF:, :],
                        preferred_element_type=jnp.float32)
        t = t + b1_ref[...]
        t_vmem[pl.ds(i * BR, BR), :] = t

        @pl.when(i == 0)
        def _():
            stats_vmem[...] = jnp.zeros_like(stats_vmem)

        s1 = jnp.sum(t, axis=0, keepdims=True)
        s2 = jnp.sum(t * t, axis=0, keepdims=True)
        stats_vmem[0:1, :] = stats_vmem[0:1, :] + s1
        stats_vmem[1:2, :] = stats_vmem[1:2, :] + s2

    @pl.when(ph == 1)
    def _():
        t = t_vmem[pl.ds(i * BR, BR), :]
        mu = stats_vmem[0:1, :] * (1.0 / N)
        msq = stats_vmem[1:2, :] * (1.0 / N)
        var = msq - mu * mu
        scale = g_ref[...] * lax.rsqrt(var + 1e-5)
        shift = be_ref[...] - mu * scale
        hbn = jnp.maximum(t * scale + shift, 0.0)
        z = jnp.dot(hbn, w2_ref[...], preferred_element_type=jnp.float32)
        z = jnp.maximum(z + b2_ref[...], 0.0)
        z_ref[0] = z[:, :HALF]
        z_ref[1] = z[:, HALF:]
        onehot = (lax.broadcasted_iota(jnp.int32, (NG, BR), 0)
                  .astype(jnp.float32) == batch_ref[0]).astype(jnp.float32)
        p = jnp.dot(onehot, z, preferred_element_type=jnp.float32)

        @pl.when(i == 0)
        def _():
            pooled_ref[...] = jnp.zeros_like(pooled_ref)

        pooled_ref[...] = pooled_ref[...] + p


_MLP_IN_SPECS = [
    pl.BlockSpec((BR, HALF), lambda p, i: (jnp.where(p == 0, i, 0), 0)),
    pl.BlockSpec((BR, HALF),
                 lambda p, i: (NBLK + jnp.where(p == 0, i, 0), 0)),
    pl.BlockSpec((H, H), lambda p, i: (0, 0)),
    pl.BlockSpec((1, H), lambda p, i: (0, 0)),
    pl.BlockSpec((1, H), lambda p, i: (0, 0)),
    pl.BlockSpec((1, H), lambda p, i: (0, 0)),
    pl.BlockSpec((H, H), lambda p, i: (0, 0)),
    pl.BlockSpec((1, H), lambda p, i: (0, 0)),
    pl.BlockSpec((1, 1, BR), lambda p, i: (i, 0, 0)),
]
_MLP_SCRATCH = [
    pltpu.VMEM((N, H), jnp.float32),
    pltpu.VMEM((8, H), jnp.float32),
]

_mlp = pl.pallas_call(
    _mlp_body,
    grid=(2, NBLK),
    in_specs=list(_MLP_IN_SPECS),
    out_specs=[
        pl.BlockSpec((2, BR, HALF), lambda p, i: (0, jnp.where(p == 1, i, 0), 0)),
        pl.BlockSpec((NG, H), lambda p, i: (0, 0)),
    ],
    out_shape=[
        jax.ShapeDtypeStruct((2, N, HALF), jnp.float32),
        jax.ShapeDtypeStruct((NG, H), jnp.float32),
    ],
    scratch_shapes=list(_MLP_SCRATCH),
)

_mlp_head = pl.pallas_call(
    _mlp_head_body,
    grid=(2, NBLK),
    in_specs=list(_MLP_IN_SPECS) + [
        pl.BlockSpec((NG, H), lambda p, i: (0, 0)),
        pl.BlockSpec((NG, H), lambda p, i: (0, 0)),
        pl.BlockSpec((NC * H, OUT), lambda p, i: (0, 0)),
        pl.BlockSpec((1, OUT), lambda p, i: (0, 0)),
    ],
    out_specs=[
        pl.BlockSpec((2, BR, HALF), lambda p, i: (0, jnp.where(p == 1, i, 0), 0)),
        pl.BlockSpec((NG, H), lambda p, i: (0, 0)),
        pl.BlockSpec((NG, OUT), lambda p, i: (0, 0)),
    ],
    out_shape=[
        jax.ShapeDtypeStruct((2, N, HALF), jnp.float32),
        jax.ShapeDtypeStruct((NG, H), jnp.float32),
        jax.ShapeDtypeStruct((NG, OUT), jnp.float32),
    ],
    scratch_shapes=list(_MLP_SCRATCH),
)


def kernel(x, edge_index, batch,
           W1_0, b1_0, g_0, be_0, W2_0, b2_0,
           W1_1, b1_1, g_1, be_1, W2_1, b2_1,
           W1_2, b1_2, g_2, be_2, W2_2, b2_2,
           Wl, bl):
    src = edge_index[0].astype(jnp.int32)
    dst = edge_index[1].astype(jnp.int32)
    src2 = jnp.stack([src, src + N]).reshape(NSC, NTILES, EPT)
    dst2 = dst.reshape(NTILES, NCH, K)
    batchf = batch.astype(jnp.float32).reshape(NBLK, 1, BR)

    h2 = jnp.concatenate([x[:, :HALF], x[:, HALF:]], axis=0)
    params = [
        (W1_0, b1_0, g_0, be_0, W2_0, b2_0),
        (W1_1, b1_1, g_1, be_1, W2_1, b2_1),
        (W1_2, b1_2, g_2, be_2, W2_2, b2_2),
    ]
    pooled = []
    for (W1, b1, g, be, W2, b2) in params[:2]:
        u2 = _sc_agg()(h2, src2, dst2)
        z2, p = _mlp(u2, u2, W1, b1.reshape(1, H),
                     g.reshape(1, H), be.reshape(1, H), W2,
                     b2.reshape(1, H), batchf)
        h2 = z2.reshape(NSC * N, HALF)
        pooled.append(p)
    (W1, b1, g, be, W2, b2) = params[2]
    u2 = _sc_agg()(h2, src2, dst2)
    _, _, out = _mlp_head(u2, u2, W1, b1.reshape(1, H),
                          g.reshape(1, H), be.reshape(1, H), W2,
                          b2.reshape(1, H), batchf,
                          pooled[0], pooled[1], Wl, bl.reshape(1, OUT))
    return out
